# baseline (speedup 1.0000x reference)
import functools
import jax
import jax.numpy as jnp
from jax import lax
from jax.experimental import pallas as pl
from jax.experimental.pallas import tpu as pltpu
from jax.experimental.pallas import tpu_sc as plsc


def kernel(inputs, pembs_weight):
    batch_size, seqs_len = inputs.shape[:2]
    num_units = pembs_weight.shape[1]
    table = pembs_weight[:seqs_len]

    NC, NS = 2, 16
    NW = NC * NS
    b_per_w = batch_size // NW

    mesh = plsc.VectorSubcoreMesh(core_axis_name="c", subcore_axis_name="s")

    @functools.partial(
        pl.kernel,
        mesh=mesh,
        out_type=jax.ShapeDtypeStruct((batch_size, seqs_len, num_units), jnp.float32),
        scratch_types=[
            pltpu.VMEM((seqs_len, num_units), jnp.float32),
            pltpu.VMEM_SHARED((seqs_len, num_units), jnp.float32),
            pltpu.SemaphoreType.DMA,
            pltpu.SemaphoreType.DMA,
        ],
    )
    def k(table_hbm, out_hbm, tab_v, tab_sh, sem_a, sem_b):
        sid = lax.axis_index("s")
        wid = sid * NC + lax.axis_index("c")
        base = wid * b_per_w
        pltpu.sync_copy(table_hbm, tab_v)

        @pl.when(sid == 0)
        def _():
            pltpu.sync_copy(table_hbm, tab_sh)

        plsc.subcore_barrier()

        def body(i, carry):
            a = pltpu.make_async_copy(tab_v, out_hbm.at[base + 2 * i], sem_a)
            b = pltpu.make_async_copy(tab_sh, out_hbm.at[base + 2 * i + 1], sem_b)
            a.start()
            b.start()
            a.wait()
            b.wait()
            return carry

        lax.fori_loop(0, b_per_w // 2, body, 0)

    return k(table)


# speedup vs baseline: 1.0488x; 1.0488x over previous
import functools
import jax
import jax.numpy as jnp
from jax import lax
from jax.experimental import pallas as pl
from jax.experimental.pallas import tpu as pltpu
from jax.experimental.pallas import tpu_sc as plsc


def kernel(inputs, pembs_weight):
    batch_size, seqs_len = inputs.shape[:2]
    num_units = pembs_weight.shape[1]
    table = pembs_weight[:seqs_len]

    NC, NS = 2, 16
    NW = NC * NS
    b_per_w = batch_size // NW

    mesh = plsc.VectorSubcoreMesh(core_axis_name="c", subcore_axis_name="s")

    @functools.partial(
        pl.kernel,
        mesh=mesh,
        out_type=jax.ShapeDtypeStruct((batch_size, seqs_len, num_units), jnp.float32),
        scratch_types=[pltpu.VMEM((seqs_len, num_units), jnp.float32)],
    )
    def k(table_hbm, out_hbm, tab_v):
        wid = lax.axis_index("s") * NC + lax.axis_index("c")
        base = wid * b_per_w
        pltpu.sync_copy(table_hbm, tab_v)

        def body(i, carry):
            pltpu.sync_copy(tab_v, out_hbm.at[base + i])
            return carry

        lax.fori_loop(0, b_per_w, body, 0)

    return k(table)
